# baseline (speedup 1.0000x reference)
import functools

import numpy as np
import jax
import jax.numpy as jnp
from jax import lax
from jax.experimental import pallas as pl
from jax.experimental.pallas import tpu as pltpu
from jax.experimental.pallas import tpu_sc as plsc

N_USERS = 5000
N_ITEMS = 5000
N = N_USERS + N_ITEMS
E = 320000
D = 128
L = 2

NS = 16
C = 64
GSZ = 5
NGRP = 63
NCH = NGRP * GSZ
EPT = NCH * C
EP = NS * EPT
NPAD = 10240
RPT = NPAD // NS
NBUF = 4
EBUF = 4
EWC = 80

_GDN = jax.lax.GatherDimensionNumbers(
    offset_dims=(), collapsed_slice_dims=(0,), start_index_map=(0,))


def _sc_two_layers(ego0f, rec4, ew3, zerosp):
    mesh = plsc.VectorSubcoreMesh(core_axis_name="c", subcore_axis_name="s",
                                  num_cores=2, num_subcores=NS)

    @functools.partial(
        pl.kernel,
        out_type=(jax.ShapeDtypeStruct((2 * NPAD, D), jnp.float32),
                  jax.ShapeDtypeStruct((2 * NPAD, D), jnp.float32)),
        mesh=mesh,
        scratch_types=[
            pltpu.VMEM((EBUF, 2 * GSZ, C), jnp.int32),
            pltpu.VMEM((EBUF, GSZ, C), jnp.float32),
            pltpu.VMEM((NBUF, C, D), jnp.float32),
            pltpu.VMEM_SHARED((NPAD, D), jnp.float32),
            pltpu.SemaphoreType.DMA((EBUF,)),
            pltpu.SemaphoreType.DMA((NBUF,)),
            pltpu.SemaphoreType.DMA((NBUF,)),
        ],
    )
    def agg(ego0_hbm, rec_hbm, ew_hbm, zero_hbm, ego1_hbm, ego2_hbm,
            ering, wring, rows, acc, esem, gsem, ssem):
        cid = lax.axis_index("c")
        sid = lax.axis_index("s")
        gr0 = sid * NGRP

        pltpu.sync_copy(zero_hbm.at[pl.ds(sid * RPT, RPT)],
                        acc.at[pl.ds(sid * RPT, RPT)])
        plsc.subcore_barrier()

        def issue_rec(g):
            b = lax.rem(g, EBUF)
            pltpu.async_copy(rec_hbm.at[cid, gr0 + g], ering.at[b],
                             esem.at[b])
            pltpu.async_copy(ew_hbm.at[cid, gr0 + g], wring.at[b],
                             esem.at[b])

        def wait_rec(g):
            b = lax.rem(g, EBUF)
            pltpu.make_async_copy(rec_hbm.at[cid, gr0 + g], ering.at[b],
                                  esem.at[b]).wait()
            pltpu.make_async_copy(ew_hbm.at[cid, gr0 + g], wring.at[b],
                                  esem.at[b]).wait()

        def layer(table_hbm, out_hbm, rezero):
            def issue_gather(k, gb, sub):
                b = lax.rem(k, NBUF)
                pltpu.async_copy(table_hbm.at[ering.at[gb, sub]],
                                 rows.at[b], gsem.at[b])

            def wait_gather(k, gb, sub):
                b = lax.rem(k, NBUF)
                pltpu.make_async_copy(table_hbm.at[ering.at[gb, sub]],
                                      rows.at[b], gsem.at[b]).wait()

            def issue_scatter(k, gb, sub):
                b = lax.rem(k, NBUF)
                pltpu.async_copy(rows.at[b], acc.at[ering.at[gb, GSZ + sub]],
                                 ssem.at[b], add=True)

            def wait_scatter_slot(bslot, gb, sub):
                pltpu.make_async_copy(rows.at[bslot],
                                      acc.at[ering.at[gb, GSZ + sub]],
                                      ssem.at[bslot]).wait()

            issue_rec(0)
            issue_rec(1)
            wait_rec(0)
            issue_gather(0, 0, 0)
            issue_gather(1, 0, 1)

            def group(g, carry):
                ge = lax.rem(g, EBUF)
                ge1 = lax.rem(g + 1, EBUF)
                gem1 = lax.rem(g + EBUF - 1, EBUF)
                k0 = g * GSZ

                @pl.when(g < NGRP - 2)
                def _():
                    issue_rec(g + 2)

                for sub in range(GSZ):
                    k = k0 + sub
                    wait_gather(k, ge, sub)
                    b = lax.rem(k, NBUF)

                    @plsc.parallel_loop(0, C // 16, 1, unroll=4)
                    def _(gg):
                        w16 = wring[ge, sub, pl.ds(gg * 16, 16)]
                        for r in range(16):
                            idx = lax.broadcast_in_dim(
                                jnp.int32(r), (16, 1), ())
                            wspl = lax.gather(
                                w16, idx, _GDN, (1,),
                                mode=lax.GatherScatterMode.PROMISE_IN_BOUNDS)
                            rr = gg * 16 + r
                            for j in range(D // 16):
                                rows[b, rr, pl.ds(j * 16, 16)] = (
                                    rows[b, rr, pl.ds(j * 16, 16)] * wspl)
                    issue_scatter(k, ge, sub)

                    if sub >= 2:
                        gb2, sub2 = ge, sub - 2
                    else:
                        gb2, sub2 = gem1, sub + GSZ - 2

                    @pl.when(k >= 2)
                    def _():
                        wait_scatter_slot(lax.rem(k - 2, NBUF), gb2, sub2)

                    if sub == 2:
                        @pl.when(g < NGRP - 1)
                        def _():
                            wait_rec(g + 1)

                    if sub < GSZ - 2:
                        gbn, subn = ge, sub + 2
                    else:
                        gbn, subn = ge1, sub + 2 - GSZ

                    @pl.when(k < NCH - 2)
                    def _():
                        issue_gather(k + 2, gbn, subn)
                return carry

            lax.fori_loop(0, NGRP, group, 0)
            wait_scatter_slot(lax.rem(NCH - 2, NBUF),
                              lax.rem(NGRP - 1, EBUF), GSZ - 2)
            wait_scatter_slot(lax.rem(NCH - 1, NBUF),
                              lax.rem(NGRP - 1, EBUF), GSZ - 1)
            plsc.subcore_barrier()

            for q in range(RPT // EWC):
                r0 = sid * RPT + q * EWC
                g0 = cid * NPAD + r0
                pltpu.sync_copy(acc.at[pl.ds(r0, EWC)],
                                rows.at[0, pl.ds(0, EWC)])
                pltpu.sync_copy(table_hbm.at[pl.ds(g0, EWC)],
                                rows.at[1, pl.ds(0, EWC)])
                if rezero:
                    pltpu.sync_copy(zero_hbm.at[pl.ds(r0, EWC)],
                                    acc.at[pl.ds(r0, EWC)])

                @plsc.parallel_loop(0, EWC, 1, unroll=2)
                def _(r):
                    for j in range(D // 16):
                        a = rows[0, r, pl.ds(j * 16, 16)]
                        e = rows[1, r, pl.ds(j * 16, 16)]
                        rows[0, r, pl.ds(j * 16, 16)] = a + e * a
                pltpu.sync_copy(rows.at[0, pl.ds(0, EWC)],
                                out_hbm.at[pl.ds(g0, EWC)])
            plsc.subcore_barrier()

        layer(ego0_hbm, ego1_hbm, rezero=True)
        layer(ego1_hbm, ego2_hbm, rezero=False)

    return agg(ego0f, rec4, ew3, zerosp)


def _proj_body(x0_ref, x1_ref, x2_ref, w_ref, b_ref, o_ref):
    x0 = x0_ref[0]
    x1 = x1_ref[0]
    x2 = x2_ref[0]
    n1 = x1 / jnp.maximum(jnp.sqrt(jnp.sum(x1 * x1, axis=1, keepdims=True)),
                          1e-12)
    n2 = x2 / jnp.maximum(jnp.sqrt(jnp.sum(x2 * x2, axis=1, keepdims=True)),
                          1e-12)
    w = w_ref[0]
    y = (jnp.dot(x0, w[0:D], preferred_element_type=jnp.float32)
         + jnp.dot(n1, w[D:2 * D], preferred_element_type=jnp.float32)
         + jnp.dot(n2, w[2 * D:3 * D], preferred_element_type=jnp.float32)
         + b_ref[0, 0])
    o_ref[...] = jnp.where(y >= 0, y, 0.01 * y)


def _tc_project(ego0p, ego1p, ego2p, w4, b4):
    br = 1000
    nb = N_USERS // br
    grid = (4, nb)
    node_spec = pl.BlockSpec(
        (1, br, D), lambda i, j: (i // 2, (i % 2) * nb + j, 0))
    return pl.pallas_call(
        _proj_body,
        grid=grid,
        in_specs=[
            node_spec, node_spec, node_spec,
            pl.BlockSpec((1, (L + 1) * D, D), lambda i, j: (i, 0, 0)),
            pl.BlockSpec((1, 1, D), lambda i, j: (i, 0, 0)),
        ],
        out_specs=pl.BlockSpec((br, D), lambda i, j: (i * nb + j, 0)),
        out_shape=jax.ShapeDtypeStruct((4 * N_USERS, D), jnp.float32),
    )(ego0p, ego1p, ego2p, w4, b4)


def kernel(user_emb_s, item_emb_s, user_emb_t, item_emb_t, ew_s, ew_t,
           W_s, b_s, W_si, b_si, W_t, b_t, W_ti, b_ti,
           src_s, dst_s, src_t, dst_t):
    zpad = jnp.zeros((NPAD - N, D), jnp.float32)
    ego0f = jnp.concatenate([user_emb_s, item_emb_s, zpad,
                             user_emb_t, item_emb_t, zpad], axis=0)

    epad_i = jnp.zeros((EP - E,), jnp.int32)
    epad_f = jnp.zeros((EP - E,), jnp.float32)
    src2 = jnp.stack([jnp.concatenate([src_s, epad_i]),
                      jnp.concatenate([src_t, epad_i]) + NPAD])
    dst2 = jnp.stack([jnp.concatenate([dst_s, epad_i]),
                      jnp.concatenate([dst_t, epad_i])])
    ew3 = jnp.stack([jnp.concatenate([ew_s, epad_f]),
                     jnp.concatenate([ew_t, epad_f])]
                    ).reshape(2, NS * NGRP, GSZ, C)
    rec4 = jnp.concatenate([src2.reshape(2, NS * NGRP, GSZ, C),
                            dst2.reshape(2, NS * NGRP, GSZ, C)], axis=2)
    zerosp = jnp.zeros((NPAD, D), jnp.float32)

    ego1f, ego2f = _sc_two_layers(ego0f, rec4, ew3, zerosp)

    w4 = jnp.stack([W_s, W_si, W_t, W_ti])
    b4 = jnp.stack([b_s, b_si, b_t, b_ti])[:, None, :]
    return _tc_project(ego0f.reshape(2, NPAD, D),
                       ego1f.reshape(2, NPAD, D),
                       ego2f.reshape(2, NPAD, D), w4, b4)


# speedup vs baseline: 1.6176x; 1.6176x over previous
import functools

import numpy as np
import jax
import jax.numpy as jnp
from jax import lax
from jax.experimental import pallas as pl
from jax.experimental.pallas import tpu as pltpu
from jax.experimental.pallas import tpu_sc as plsc

N_USERS = 5000
N_ITEMS = 5000
N = N_USERS + N_ITEMS
E = 320000
D = 128
L = 2

NS = 16
C = 80
NCH = 250
EPT = NCH * C
EP = NS * EPT
NPAD = 10240
RPT = NPAD // NS
NBUF = 4
EBUF = 8
EWC = 80

_GDN = jax.lax.GatherDimensionNumbers(
    offset_dims=(), collapsed_slice_dims=(0,), start_index_map=(0,))


def _sc_two_layers(ego0f, rec4, ew3, zerosp):
    mesh = plsc.VectorSubcoreMesh(core_axis_name="c", subcore_axis_name="s",
                                  num_cores=2, num_subcores=NS)

    @functools.partial(
        pl.kernel,
        out_type=(jax.ShapeDtypeStruct((2 * NPAD, D), jnp.float32),
                  jax.ShapeDtypeStruct((2 * NPAD, D), jnp.float32)),
        mesh=mesh,
        scratch_types=[
            pltpu.VMEM((EBUF, 2, C), jnp.int32),
            pltpu.VMEM((EBUF, C), jnp.float32),
            pltpu.VMEM((NBUF, C, D), jnp.float32),
            pltpu.VMEM_SHARED((NPAD, D), jnp.float32),
            pltpu.SemaphoreType.DMA((EBUF,)),
            pltpu.SemaphoreType.DMA((NBUF,)),
            pltpu.SemaphoreType.DMA((NBUF,)),
        ],
    )
    def agg(ego0_hbm, rec_hbm, ew_hbm, zero_hbm, ego1_hbm, ego2_hbm,
            ering, wring, rows, acc, esem, gsem, ssem):
        cid = lax.axis_index("c")
        sid = lax.axis_index("s")
        ch0 = sid * NCH

        pltpu.sync_copy(zero_hbm.at[pl.ds(sid * RPT, RPT)],
                        acc.at[pl.ds(sid * RPT, RPT)])
        plsc.subcore_barrier()

        def issue_rec(k):
            b = lax.rem(k, EBUF)
            pltpu.async_copy(rec_hbm.at[cid, ch0 + k], ering.at[b],
                             esem.at[b])
            pltpu.async_copy(ew_hbm.at[cid, ch0 + k], wring.at[b],
                             esem.at[b])

        def wait_rec(k):
            b = lax.rem(k, EBUF)
            pltpu.make_async_copy(rec_hbm.at[cid, ch0 + k], ering.at[b],
                                  esem.at[b]).wait()
            pltpu.make_async_copy(ew_hbm.at[cid, ch0 + k], wring.at[b],
                                  esem.at[b]).wait()

        def layer(table_hbm, out_hbm, rezero):
            def issue_gather(k):
                b = lax.rem(k, NBUF)
                e = lax.rem(k, EBUF)
                pltpu.async_copy(table_hbm.at[ering.at[e, 0]],
                                 rows.at[b], gsem.at[b])

            def wait_gather(k):
                b = lax.rem(k, NBUF)
                e = lax.rem(k, EBUF)
                pltpu.make_async_copy(table_hbm.at[ering.at[e, 0]],
                                      rows.at[b], gsem.at[b]).wait()

            def issue_scatter(k):
                b = lax.rem(k, NBUF)
                e = lax.rem(k, EBUF)
                pltpu.async_copy(rows.at[b], acc.at[ering.at[e, 1]],
                                 ssem.at[b], add=True)

            def wait_scatter(k):
                b = lax.rem(k, NBUF)
                e = lax.rem(k, EBUF)
                pltpu.make_async_copy(rows.at[b], acc.at[ering.at[e, 1]],
                                      ssem.at[b]).wait()

            for kk in range(6):
                issue_rec(kk)
            wait_rec(0)
            issue_gather(0)
            wait_rec(1)
            issue_gather(1)

            def chunk(k, carry):
                wait_gather(k)
                b = lax.rem(k, NBUF)
                e = lax.rem(k, EBUF)

                @plsc.parallel_loop(0, C // 16, 1, unroll=4)
                def _(g):
                    w16 = wring[e, pl.ds(g * 16, 16)]
                    for r in range(16):
                        idx = lax.broadcast_in_dim(
                            jnp.int32(r), (16, 1), ())
                        wspl = lax.gather(
                            w16, idx, _GDN, (1,),
                            mode=lax.GatherScatterMode.PROMISE_IN_BOUNDS)
                        rr = g * 16 + r
                        for j in range(D // 16):
                            rows[b, rr, pl.ds(j * 16, 16)] = (
                                rows[b, rr, pl.ds(j * 16, 16)] * wspl)
                issue_scatter(k)

                @pl.when(k >= 2)
                def _():
                    wait_scatter(k - 2)

                @pl.when(k < NCH - 2)
                def _():
                    wait_rec(k + 2)
                    issue_gather(k + 2)

                @pl.when(k < NCH - 6)
                def _():
                    issue_rec(k + 6)
                return carry

            lax.fori_loop(0, NCH, chunk, 0)
            wait_scatter(NCH - 2)
            wait_scatter(NCH - 1)
            plsc.subcore_barrier()

            for q in range(RPT // EWC):
                r0 = sid * RPT + q * EWC
                g0 = cid * NPAD + r0
                pltpu.sync_copy(acc.at[pl.ds(r0, EWC)],
                                rows.at[0, pl.ds(0, EWC)])
                pltpu.sync_copy(table_hbm.at[pl.ds(g0, EWC)],
                                rows.at[1, pl.ds(0, EWC)])
                if rezero:
                    pltpu.sync_copy(zero_hbm.at[pl.ds(r0, EWC)],
                                    acc.at[pl.ds(r0, EWC)])

                @plsc.parallel_loop(0, EWC, 1, unroll=2)
                def _(r):
                    for j in range(D // 16):
                        a = rows[0, r, pl.ds(j * 16, 16)]
                        e = rows[1, r, pl.ds(j * 16, 16)]
                        rows[0, r, pl.ds(j * 16, 16)] = a + e * a
                pltpu.sync_copy(rows.at[0, pl.ds(0, EWC)],
                                out_hbm.at[pl.ds(g0, EWC)])
            plsc.subcore_barrier()

        layer(ego0_hbm, ego1_hbm, rezero=True)
        layer(ego1_hbm, ego2_hbm, rezero=False)

    return agg(ego0f, rec4, ew3, zerosp)


def _proj_body(x0_ref, x1_ref, x2_ref, w_ref, b_ref, o_ref):
    x0 = x0_ref[0]
    x1 = x1_ref[0]
    x2 = x2_ref[0]
    n1 = x1 / jnp.maximum(jnp.sqrt(jnp.sum(x1 * x1, axis=1, keepdims=True)),
                          1e-12)
    n2 = x2 / jnp.maximum(jnp.sqrt(jnp.sum(x2 * x2, axis=1, keepdims=True)),
                          1e-12)
    w = w_ref[0]
    y = (jnp.dot(x0, w[0:D], preferred_element_type=jnp.float32)
         + jnp.dot(n1, w[D:2 * D], preferred_element_type=jnp.float32)
         + jnp.dot(n2, w[2 * D:3 * D], preferred_element_type=jnp.float32)
         + b_ref[0, 0])
    o_ref[...] = jnp.where(y >= 0, y, 0.01 * y)


def _tc_project(ego0p, ego1p, ego2p, w4, b4):
    br = 1000
    nb = N_USERS // br
    grid = (4, nb)
    node_spec = pl.BlockSpec(
        (1, br, D), lambda i, j: (i // 2, (i % 2) * nb + j, 0))
    return pl.pallas_call(
        _proj_body,
        grid=grid,
        in_specs=[
            node_spec, node_spec, node_spec,
            pl.BlockSpec((1, (L + 1) * D, D), lambda i, j: (i, 0, 0)),
            pl.BlockSpec((1, 1, D), lambda i, j: (i, 0, 0)),
        ],
        out_specs=pl.BlockSpec((br, D), lambda i, j: (i * nb + j, 0)),
        out_shape=jax.ShapeDtypeStruct((4 * N_USERS, D), jnp.float32),
    )(ego0p, ego1p, ego2p, w4, b4)


def kernel(user_emb_s, item_emb_s, user_emb_t, item_emb_t, ew_s, ew_t,
           W_s, b_s, W_si, b_si, W_t, b_t, W_ti, b_ti,
           src_s, dst_s, src_t, dst_t):
    zpad = jnp.zeros((NPAD - N, D), jnp.float32)
    ego0f = jnp.concatenate([user_emb_s, item_emb_s, zpad,
                             user_emb_t, item_emb_t, zpad], axis=0)

    epad_i = jnp.zeros((EP - E,), jnp.int32)
    epad_f = jnp.zeros((EP - E,), jnp.float32)
    src2 = jnp.stack([jnp.concatenate([src_s, epad_i]),
                      jnp.concatenate([src_t, epad_i]) + NPAD])
    dst2 = jnp.stack([jnp.concatenate([dst_s, epad_i]),
                      jnp.concatenate([dst_t, epad_i])])
    ew3 = jnp.stack([jnp.concatenate([ew_s, epad_f]),
                     jnp.concatenate([ew_t, epad_f])]).reshape(2, NS * NCH, C)
    rec4 = jnp.stack([src2.reshape(2, NS * NCH, C),
                      dst2.reshape(2, NS * NCH, C)], axis=2)
    zerosp = jnp.zeros((NPAD, D), jnp.float32)

    ego1f, ego2f = _sc_two_layers(ego0f, rec4, ew3, zerosp)

    w4 = jnp.stack([W_s, W_si, W_t, W_ti])
    b4 = jnp.stack([b_s, b_si, b_t, b_ti])[:, None, :]
    return _tc_project(ego0f.reshape(2, NPAD, D),
                       ego1f.reshape(2, NPAD, D),
                       ego2f.reshape(2, NPAD, D), w4, b4)
